# Initial kernel scaffold; baseline (speedup 1.0000x reference)
import jax
import jax.numpy as jnp
from jax.experimental import pallas as pl


def kernel(x, edge_index, W1, b1, W2, b2, W6, b6, W3, b3, W4, b4, Wl, bl):
    raise NotImplementedError("write your pallas kernel here")


# speedup vs baseline: 10.0126x; 10.0126x over previous
import functools

import jax
import jax.numpy as jnp
from jax import lax
from jax.experimental import pallas as pl
from jax.experimental.pallas import tpu as pltpu
from jax.experimental.pallas import tpu_sc as plsc

N = 10000
E = 320000
NP = 10112
DUMMY = 10000
NW = 32
CHUNK = 128
NCHUNK = 80
EP = CHUNK * NCHUNK * NW
RPT = NP // 16
R = 2528
G = NP // R

_mesh = functools.partial(plsc.VectorSubcoreMesh,
                          core_axis_name="c", subcore_axis_name="s",
                          num_cores=2, num_subcores=16)


def _edge_agg(table, srcs, dsts, zeros, dp):
  def body(table_ref, srcs_ref, dsts_ref, zeros_ref, out_ref,
           srcv, dstv, rows, agg):
    cid = lax.axis_index("c")
    sid = lax.axis_index("s")
    wid = cid * 16 + sid
    rbase = sid * RPT
    pltpu.sync_copy(zeros_ref.at[pl.ds(rbase, RPT)], agg.at[pl.ds(rbase, RPT)])
    pltpu.sync_copy(srcs_ref.at[wid], srcv)
    pltpu.sync_copy(dsts_ref.at[wid], dstv)
    plsc.subcore_barrier()

    def step(j, c):
      pltpu.sync_copy(table_ref.at[srcv.at[j]], rows)
      pltpu.sync_copy(rows, agg.at[dstv.at[j]], add=True)
      return c

    lax.fori_loop(0, NCHUNK, step, 0)
    plsc.subcore_barrier()
    pltpu.sync_copy(agg.at[pl.ds(rbase, RPT)],
                    out_ref.at[cid, pl.ds(rbase, RPT)])

  f = pl.kernel(
      body,
      out_type=jax.ShapeDtypeStruct((2, NP, dp), jnp.float32),
      mesh=_mesh(),
      compiler_params=pltpu.CompilerParams(use_tc_tiling_on_sc=False),
      scratch_types=[
          pltpu.VMEM((NCHUNK, CHUNK), jnp.int32),
          pltpu.VMEM((NCHUNK, CHUNK), jnp.int32),
          pltpu.VMEM((CHUNK, dp), jnp.float32),
          pltpu.VMEM_SHARED((NP, dp), jnp.float32),
      ])
  return f(table, srcs, dsts, zeros)


def _deg_count(dsts, ones_rows, zeros):
  def body(dsts_ref, ones_ref, zeros_ref, out_ref, dstv, rows, agg):
    cid = lax.axis_index("c")
    sid = lax.axis_index("s")
    wid = cid * 16 + sid
    rbase = sid * RPT
    pltpu.sync_copy(zeros_ref.at[pl.ds(rbase, RPT)], agg.at[pl.ds(rbase, RPT)])
    pltpu.sync_copy(dsts_ref.at[wid], dstv)
    pltpu.sync_copy(ones_ref, rows)
    plsc.subcore_barrier()

    def step(j, c):
      pltpu.sync_copy(rows, agg.at[dstv.at[j]], add=True)
      return c

    lax.fori_loop(0, NCHUNK, step, 0)
    plsc.subcore_barrier()
    pltpu.sync_copy(agg.at[pl.ds(rbase, RPT)],
                    out_ref.at[cid, pl.ds(rbase, RPT)])

  f = pl.kernel(
      body,
      out_type=jax.ShapeDtypeStruct((2, NP, 16), jnp.float32),
      mesh=_mesh(),
      scratch_types=[
          pltpu.VMEM((NCHUNK, CHUNK), jnp.int32),
          pltpu.VMEM((CHUNK, 16), jnp.float32),
          pltpu.VMEM_SHARED((NP, 16), jnp.float32),
      ])
  return f(dsts, ones_rows, zeros)


def _tc_first(degp, xp, w, dpo):
  def body(degp_ref, x_ref, w_ref, hp_ref, dinv_ref):
    deg = degp_ref[0, :, 0:1] + degp_ref[1, :, 0:1] + 1.0
    dinv = lax.rsqrt(deg)
    h = jnp.dot(x_ref[...], w_ref[...], preferred_element_type=jnp.float32)
    hp_ref[...] = h * dinv
    dinv_ref[...] = dinv

  dpi = xp.shape[1]
  return pl.pallas_call(
      body,
      grid=(G,),
      in_specs=[
          pl.BlockSpec((2, R, 16), lambda i: (0, i, 0)),
          pl.BlockSpec((R, dpi), lambda i: (i, 0)),
          pl.BlockSpec((dpi, dpo), lambda i: (0, 0)),
      ],
      out_specs=[
          pl.BlockSpec((R, dpo), lambda i: (i, 0)),
          pl.BlockSpec((R, 1), lambda i: (i, 0)),
      ],
      out_shape=[
          jax.ShapeDtypeStruct((NP, dpo), jnp.float32),
          jax.ShapeDtypeStruct((NP, 1), jnp.float32),
      ])(degp, xp, w)


def _tc_mid(p, hp, dinv, b8, w, dpo):
  def body(p_ref, hp_ref, dinv_ref, b_ref, w_ref, o_ref):
    dinv = dinv_ref[...]
    a = (p_ref[0] + p_ref[1] + hp_ref[...]) * dinv + b_ref[0:1, :]
    a = jnp.maximum(a, 0.0)
    h = jnp.dot(a, w_ref[...], preferred_element_type=jnp.float32)
    o_ref[...] = h * dinv

  dpi = hp.shape[1]
  return pl.pallas_call(
      body,
      grid=(G,),
      in_specs=[
          pl.BlockSpec((2, R, dpi), lambda i: (0, i, 0)),
          pl.BlockSpec((R, dpi), lambda i: (i, 0)),
          pl.BlockSpec((R, 1), lambda i: (i, 0)),
          pl.BlockSpec((8, dpi), lambda i: (0, 0)),
          pl.BlockSpec((dpi, dpo), lambda i: (0, 0)),
      ],
      out_specs=pl.BlockSpec((R, dpo), lambda i: (i, 0)),
      out_shape=jax.ShapeDtypeStruct((NP, dpo), jnp.float32),
  )(p, hp, dinv, b8, w)


def _tc_last(p, hp, dinv, b8, w, bl8, dpo):
  def body(p_ref, hp_ref, dinv_ref, b_ref, w_ref, bl_ref, o_ref):
    dinv = dinv_ref[...]
    a = (p_ref[0] + p_ref[1] + hp_ref[...]) * dinv + b_ref[0:1, :]
    a = jnp.maximum(a, 0.0)
    h = jnp.dot(a, w_ref[...], preferred_element_type=jnp.float32)
    o_ref[...] = h + bl_ref[0:1, :]

  dpi = hp.shape[1]
  return pl.pallas_call(
      body,
      grid=(G,),
      in_specs=[
          pl.BlockSpec((2, R, dpi), lambda i: (0, i, 0)),
          pl.BlockSpec((R, dpi), lambda i: (i, 0)),
          pl.BlockSpec((R, 1), lambda i: (i, 0)),
          pl.BlockSpec((8, dpi), lambda i: (0, 0)),
          pl.BlockSpec((dpi, dpo), lambda i: (0, 0)),
          pl.BlockSpec((8, dpo), lambda i: (0, 0)),
      ],
      out_specs=pl.BlockSpec((R, dpo), lambda i: (i, 0)),
      out_shape=jax.ShapeDtypeStruct((NP, dpo), jnp.float32),
  )(p, hp, dinv, b8, w, bl8)


def _padw(w, ri, ci):
  return jnp.pad(w, ((0, ri - w.shape[0]), (0, ci - w.shape[1])))


def _padb(b, ci):
  return jnp.tile(jnp.pad(b, (0, ci - b.shape[0]))[None, :], (8, 1))


def kernel(x, edge_index, W1, b1, W2, b2, W6, b6, W3, b3, W4, b4, Wl, bl):
  src = edge_index[0]
  dst = edge_index[1]
  npad = EP - E
  srcp = jnp.concatenate(
      [src, jnp.zeros((npad,), jnp.int32)]).reshape(NW, NCHUNK, CHUNK)
  dstp = jnp.concatenate(
      [dst, jnp.full((npad,), DUMMY, jnp.int32)]).reshape(NW, NCHUNK, CHUNK)
  xp = jnp.pad(x, ((0, NP - N), (0, 0)))

  W1p = _padw(W1, 128, 96)
  W2p = _padw(W2, 96, 80)
  W6p = _padw(W6, 80, 64)
  W3p = _padw(W3, 64, 32)
  W4p = _padw(W4, 32, 32)
  Wlp = _padw(Wl, 32, 128)
  b1p = _padb(b1, 96)
  b2p = _padb(b2, 80)
  b6p = _padb(b6, 64)
  b3p = _padb(b3, 32)
  b4p = _padb(b4, 32)
  blp = _padb(bl, 128)

  ones16 = jnp.ones((CHUNK, 16), jnp.float32)
  z16 = jnp.zeros((NP, 16), jnp.float32)
  z96 = jnp.zeros((NP, 96), jnp.float32)
  z80 = jnp.zeros((NP, 80), jnp.float32)
  z64 = jnp.zeros((NP, 64), jnp.float32)
  z32 = jnp.zeros((NP, 32), jnp.float32)

  degp = _deg_count(dstp, ones16, z16)
  hp1, dinv = _tc_first(degp, xp, W1p, 96)
  p1 = _edge_agg(hp1, srcp, dstp, z96, 96)
  hp2 = _tc_mid(p1, hp1, dinv, b1p, W2p, 80)
  p2 = _edge_agg(hp2, srcp, dstp, z80, 80)
  hp3 = _tc_mid(p2, hp2, dinv, b2p, W6p, 64)
  p3 = _edge_agg(hp3, srcp, dstp, z64, 64)
  hp4 = _tc_mid(p3, hp3, dinv, b6p, W3p, 32)
  p4 = _edge_agg(hp4, srcp, dstp, z32, 32)
  hp5 = _tc_mid(p4, hp4, dinv, b3p, W4p, 32)
  p5 = _edge_agg(hp5, srcp, dstp, z32, 32)
  outp = _tc_last(p5, hp5, dinv, b4p, Wlp, blp, 128)
  return outp[:N, :4]
